# baseline (device time: 241581 ns/iter reference)
import jax
import jax.numpy as jnp
from jax import lax
from jax.experimental import pallas as pl
from jax.experimental.pallas import tpu as pltpu

N_DEV = 32


def kernel(x, w_mat):
    m, k = x.shape
    k2, n = w_mat.shape
    m_per = m // N_DEV

    def body(x_ref, w_ref, out_ref, part_ref, comm_ref, send_sems, recv_sems):
        d = lax.axis_index("i")
        left = lax.rem(d + N_DEV - 1, N_DEV)
        right = lax.rem(d + 1, N_DEV)

        barrier_sem = pltpu.get_barrier_semaphore()
        for nbr in (left, right):
            pl.semaphore_signal(
                barrier_sem, inc=1,
                device_id=(nbr,), device_id_type=pl.DeviceIdType.MESH,
            )
        pl.semaphore_wait(barrier_sem, 2)

        part_ref[:, :] = jnp.dot(
            x_ref[:, :], w_ref[:, :], preferred_element_type=jnp.float32
        )

        c0 = lax.rem(d + N_DEV - 1, N_DEV)
        comm_ref[0, :, :] = part_ref[pl.ds(c0 * m_per, m_per), :]

        for h in range(N_DEV - 1):
            s = h % 2
            r = (h + 1) % 2
            rdma = pltpu.make_async_remote_copy(
                src_ref=comm_ref.at[s],
                dst_ref=comm_ref.at[r],
                send_sem=send_sems.at[s],
                recv_sem=recv_sems.at[r],
                device_id=(right,),
                device_id_type=pl.DeviceIdType.MESH,
            )
            rdma.start()
            rdma.wait()
            c = lax.rem(d + 2 * N_DEV - 2 - h, N_DEV)
            comm_ref[r, :, :] = (
                comm_ref[r, :, :] + part_ref[pl.ds(c * m_per, m_per), :]
            )

        acc = comm_ref[(N_DEV - 1) % 2, :, :]
        out_ref[:, :] = acc * jax.nn.sigmoid(acc)

    return pl.pallas_call(
        body,
        out_shape=jax.ShapeDtypeStruct((m_per, n), jnp.float32),
        in_specs=[
            pl.BlockSpec(memory_space=pltpu.VMEM),
            pl.BlockSpec(memory_space=pltpu.VMEM),
        ],
        out_specs=pl.BlockSpec(memory_space=pltpu.VMEM),
        scratch_shapes=[
            pltpu.VMEM((m, n), jnp.float32),
            pltpu.VMEM((2, m_per, n), jnp.float32),
            pltpu.SemaphoreType.DMA((2,)),
            pltpu.SemaphoreType.DMA((2,)),
        ],
        compiler_params=pltpu.CompilerParams(collective_id=0),
    )(x, w_mat)


# device time: 179741 ns/iter; 1.3441x vs baseline; 1.3441x over previous
import jax
import jax.numpy as jnp
from jax import lax
from jax.experimental import pallas as pl
from jax.experimental.pallas import tpu as pltpu

N_DEV = 32


def kernel(x, w_mat):
    m, k = x.shape
    k2, n = w_mat.shape
    m_per = m // N_DEV
    h_per = m_per // 2

    def body(x_ref, w_ref, out_ref, part_ref,
             comm_a, comm_b, send_a, recv_a, send_b, recv_b):
        d = lax.axis_index("i")
        left = lax.rem(d + N_DEV - 1, N_DEV)
        right = lax.rem(d + 1, N_DEV)

        barrier_sem = pltpu.get_barrier_semaphore()
        for nbr in (left, right):
            pl.semaphore_signal(
                barrier_sem, inc=1,
                device_id=(nbr,), device_id_type=pl.DeviceIdType.MESH,
            )
        pl.semaphore_wait(barrier_sem, 2)

        part_ref[:, :] = jnp.dot(
            x_ref[:, :], w_ref[:, :], preferred_element_type=jnp.float32
        )

        def top(c):
            return part_ref[pl.ds(c * m_per, h_per), :]

        def bot(c):
            return part_ref[pl.ds(c * m_per + h_per, h_per), :]

        c0a = lax.rem(d + N_DEV - 1, N_DEV)
        c0b = lax.rem(d + 1, N_DEV)
        comm_a[0, :, :] = top(c0a).astype(jnp.bfloat16)
        comm_b[0, :, :] = bot(c0b).astype(jnp.bfloat16)

        for h in range(N_DEV - 1):
            s = h % 2
            r = (h + 1) % 2
            rdma_a = pltpu.make_async_remote_copy(
                src_ref=comm_a.at[s],
                dst_ref=comm_a.at[r],
                send_sem=send_a.at[s],
                recv_sem=recv_a.at[r],
                device_id=(right,),
                device_id_type=pl.DeviceIdType.MESH,
            )
            rdma_b = pltpu.make_async_remote_copy(
                src_ref=comm_b.at[s],
                dst_ref=comm_b.at[r],
                send_sem=send_b.at[s],
                recv_sem=recv_b.at[r],
                device_id=(left,),
                device_id_type=pl.DeviceIdType.MESH,
            )
            rdma_a.start()
            rdma_b.start()
            rdma_a.wait()
            rdma_b.wait()

            ca = lax.rem(d + 2 * N_DEV - 2 - h, N_DEV)
            cb = lax.rem(d + 2 + h, N_DEV)
            if h < N_DEV - 2:
                comm_a[r, :, :] = (comm_a[r, :, :] + top(ca)).astype(jnp.bfloat16)
                comm_b[r, :, :] = (comm_b[r, :, :] + bot(cb)).astype(jnp.bfloat16)
            else:
                acc_t = comm_a[r, :, :].astype(jnp.float32) + top(ca)
                acc_b = comm_b[r, :, :].astype(jnp.float32) + bot(cb)
                out_ref[0:h_per, :] = acc_t * jax.nn.sigmoid(acc_t)
                out_ref[h_per:m_per, :] = acc_b * jax.nn.sigmoid(acc_b)

    return pl.pallas_call(
        body,
        out_shape=jax.ShapeDtypeStruct((m_per, n), jnp.float32),
        in_specs=[
            pl.BlockSpec(memory_space=pltpu.VMEM),
            pl.BlockSpec(memory_space=pltpu.VMEM),
        ],
        out_specs=pl.BlockSpec(memory_space=pltpu.VMEM),
        scratch_shapes=[
            pltpu.VMEM((m, n), jnp.float32),
            pltpu.VMEM((2, h_per, n), jnp.bfloat16),
            pltpu.VMEM((2, h_per, n), jnp.bfloat16),
            pltpu.SemaphoreType.DMA((2,)),
            pltpu.SemaphoreType.DMA((2,)),
            pltpu.SemaphoreType.DMA((2,)),
            pltpu.SemaphoreType.DMA((2,)),
        ],
        compiler_params=pltpu.CompilerParams(collective_id=0),
    )(x, w_mat)
